# baseline (speedup 1.0000x reference)
import functools

import jax
import jax.numpy as jnp
from jax import lax
from jax.experimental import pallas as pl
from jax.experimental.pallas import tpu as pltpu
from jax.experimental.pallas import tpu_sc as plsc

NC = 2
NS = 16
NW = NC * NS
NP = 2
NV = NW * NP
L = 16
GB = 128


def _prop_kernel(N_FULL, CW, NBLK, K):
    mesh = plsc.VectorSubcoreMesh(
        core_axis_name="c", subcore_axis_name="s", num_cores=NC)

    @functools.partial(
        pl.kernel,
        out_type=jax.ShapeDtypeStruct((K + 1, NV, N_FULL * CW), jnp.float32),
        mesh=mesh,
        compiler_params=pltpu.CompilerParams(needs_layout_passes=False),
        scratch_types=[
            pltpu.VMEM((2, GB, L), jnp.int32),
            pltpu.VMEM((2, GB, L), jnp.int32),
            pltpu.VMEM((N_FULL * CW,), jnp.float32),
            pltpu.VMEM((N_FULL * CW,), jnp.float32),
            pltpu.SemaphoreType.DMA,
            pltpu.SemaphoreType.DMA,
        ],
    )
    def prop(xs_hbm, src_hbm, dst_hbm, out_hbm,
             src_blk, dst_blk, hb0, hb1, i0, i1):
        cid = lax.axis_index("c")
        s = lax.axis_index("s")
        w = cid * NS + s
        isem = (i0, i1)

        def iprefetch(ib, b):
            pltpu.async_copy(src_hbm.at[ib], src_blk.at[b], isem[b])
            pltpu.async_copy(dst_hbm.at[ib], dst_blk.at[b], isem[b])

        def iwait(b):
            pltpu.make_async_copy(
                src_hbm.at[0], src_blk.at[b], isem[b]).wait()
            pltpu.make_async_copy(
                dst_hbm.at[0], dst_blk.at[b], isem[b]).wait()

        for p in range(NP):
            vw = w * NP + p

            pltpu.sync_copy(xs_hbm.at[vw], hb0)
            pltpu.sync_copy(xs_hbm.at[vw], out_hbm.at[0].at[vw])

            for k in range(K):
                h_ref = hb0 if k % 2 == 0 else hb1
                a_ref = hb1 if k % 2 == 0 else hb0

                def zrow(i, c2):
                    a_ref[pl.ds(i * 16, 16)] = jnp.zeros((16,), jnp.float32)
                    return c2
                lax.fori_loop(0, N_FULL * CW // 16, zrow, 0)

                def do_block(ib, b):
                    iwait(b)

                    def grouppair(q, c2):
                        for g in (2 * q, 2 * q + 1):
                            sv = src_blk[b, g, :] * CW
                            dv = dst_blk[b, g, :] * CW
                            for j in range(CW):
                                v = plsc.load_gather(h_ref, [sv + j])
                                plsc.addupdate_scatter(a_ref, [dv + j], v)
                        return c2
                    lax.fori_loop(0, GB // 2, grouppair, 0)

                    @pl.when(ib + 2 < NBLK)
                    def _():
                        iprefetch(ib + 2, b)

                iprefetch(0, 0)
                iprefetch(1, 1)

                def blockpair(bp, c2):
                    ib = 2 * bp
                    do_block(ib, 0)
                    do_block(ib + 1, 1)
                    return c2
                lax.fori_loop(0, NBLK // 2, blockpair, 0)

                pltpu.sync_copy(a_ref, out_hbm.at[k + 1].at[vw])

    return prop


def _att_sum_kernel(hs_ref, att_ref, out_ref):
    acc = att_ref[0] * hs_ref[0]
    for k in range(1, hs_ref.shape[0]):
        acc = acc + att_ref[k] * hs_ref[k]
    out_ref[...] = acc


def kernel(x, edge_index, att):
    N, D = x.shape
    E = edge_index.shape[1]
    K = att.shape[0] - 1
    CW = D // NV

    N_FULL = ((N + 8) // 8) * 8 + 16
    blk = 2 * GB * L
    E_pad = ((E + blk - 1) // blk) * blk
    NBLK = E_pad // (GB * L)

    src = jnp.concatenate(
        [edge_index[0], jnp.zeros((E_pad - E,), jnp.int32)]
    ).reshape(NBLK, GB, L)
    dst = jnp.concatenate(
        [edge_index[1], jnp.full((E_pad - E,), N, jnp.int32)]
    ).reshape(NBLK, GB, L)

    x_full = jnp.pad(x, ((0, N_FULL - N), (0, 0)))
    xs = jnp.transpose(
        x_full.reshape(N_FULL, NV, CW), (1, 0, 2)).reshape(NV, N_FULL * CW)

    hs = _prop_kernel(N_FULL, CW, NBLK, K)(xs, src, dst)

    BR = 8
    res = pl.pallas_call(
        _att_sum_kernel,
        grid=(NV // BR,),
        in_specs=[
            pl.BlockSpec((K + 1, BR, N_FULL * CW), lambda i: (0, i, 0)),
            pl.BlockSpec(memory_space=pltpu.SMEM),
        ],
        out_specs=pl.BlockSpec((BR, N_FULL * CW), lambda i: (i, 0)),
        out_shape=jax.ShapeDtypeStruct((NV, N_FULL * CW), jnp.float32),
    )(hs, att)
    out_full = jnp.transpose(
        res.reshape(NV, N_FULL, CW), (1, 0, 2)).reshape(N_FULL, D)
    return out_full[:N]
